# baseline (speedup 1.0000x reference)
import functools

import jax
import jax.numpy as jnp
import numpy as np
from jax import lax
from jax.experimental import pallas as pl
from jax.experimental.pallas import tpu as pltpu
from jax.experimental.pallas import tpu_sc as plsc

_F = 26
_FIELD = 1000
_TOTAL = _F * _FIELD
_D = 16
_B = 4096
_P = (_F * (_F - 1)) // 2
_G = 4
_S = 32
_BLK = _G * _S
_NW = 32
_EPW = _B // _NW
_NBLK = _EPW // _G
_VROWS = _F * _BLK


def _build_tables():
    offt = np.zeros(_BLK, np.int32)
    for n in range(_BLK):
        k = n % _S
        offt[n] = k * _FIELD if k < _F else 0
    arow = np.zeros(336, np.int32)
    brow = np.zeros(336, np.int32)
    p = 0
    for i in range(_F - 1):
        for j in range(i + 1, _F):
            arow[p] = j * _BLK + i
            brow[p] = i * _BLK + j
            p += 1
    return offt, arow, brow


_OFFT, _AROW, _BROW = _build_tables()

_mesh = plsc.VectorSubcoreMesh(
    core_axis_name="c", subcore_axis_name="s", num_cores=2, num_subcores=16
)


@functools.partial(
    pl.kernel,
    out_type=jax.ShapeDtypeStruct((_B,), jnp.float32),
    mesh=_mesh,
    scratch_types=[
        pltpu.VMEM((_EPW * _S,), jnp.int32),
        pltpu.VMEM((_BLK,), jnp.int32),
        pltpu.VMEM((336,), jnp.int32),
        pltpu.VMEM((336,), jnp.int32),
        pltpu.VMEM((2, _BLK), jnp.int32),
        pltpu.VMEM((2, _VROWS, _D), jnp.float32),
        pltpu.VMEM((2, _BLK, _D), jnp.float32),
        pltpu.VMEM((_EPW,), jnp.float32),
        pltpu.SemaphoreType.DMA,
        pltpu.SemaphoreType.DMA,
    ],
    compiler_params=pltpu.CompilerParams(
        needs_layout_passes=False, use_tc_tiling_on_sc=False
    ),
)
def _ffm_sc(x_hbm, e3_hbm, w16_hbm, offt_hbm, ar_hbm, br_hbm, out_hbm,
            x_v, offt_v, ar_v, br_v, idx_v, v_v, wr_v, res_v, sem0, sem1):
    wid = lax.axis_index("s") * 2 + lax.axis_index("c")
    base = wid * _EPW

    pltpu.sync_copy(x_hbm.at[pl.ds(base * _S, _EPW * _S)], x_v)
    pltpu.sync_copy(offt_hbm, offt_v)
    pltpu.sync_copy(ar_hbm, ar_v)
    pltpu.sync_copy(br_hbm, br_v)

    sems = (sem0, sem1)

    def build(bb, buf):
        for c in range(_BLK // _D):
            xv = x_v[pl.ds(bb * _BLK + c * _D, _D)]
            ov = offt_v[pl.ds(c * _D, _D)]
            idx_v[buf, pl.ds(c * _D, _D)] = xv + ov

    def fire(buf):
        sem = sems[buf]
        ir = idx_v.at[buf]
        for j in range(_F):
            pltpu.async_copy(
                e3_hbm.at[j].at[ir], v_v.at[buf, pl.ds(j * _BLK, _BLK)], sem
            )
        pltpu.async_copy(w16_hbm.at[ir], wr_v.at[buf], sem)

    def drain(buf):
        sem = sems[buf]
        ir = idx_v.at[buf]
        for j in range(_F):
            pltpu.make_async_copy(
                e3_hbm.at[j].at[ir], v_v.at[buf, pl.ds(j * _BLK, _BLK)], sem
            ).wait()
        pltpu.make_async_copy(w16_hbm.at[ir], wr_v.at[buf], sem).wait()

    def compute(bb, buf):
        for g in range(_G):
            gb = g * _S

            def pair_body(p16, acc):
                q = p16 * _D
                arv = ar_v[pl.ds(q, _D)]
                brv = br_v[pl.ds(q, _D)]
                for u in range(_D):
                    acc = acc + v_v[buf, arv[u] + gb, :] * v_v[buf, brv[u] + gb, :]
                return acc

            acc = lax.fori_loop(
                0, _P // _D, pair_body, jnp.zeros((_D,), jnp.float32)
            )
            arv = ar_v[pl.ds(320, _D)]
            brv = br_v[pl.ds(320, _D)]
            for u in range(_P - (_P // _D) * _D):
                acc = acc + v_v[buf, arv[u] + gb, :] * v_v[buf, brv[u] + gb, :]

            def lin_body(k2, accl):
                k = k2 * 2
                return (
                    accl
                    + wr_v[buf, gb + k, :]
                    + wr_v[buf, gb + k + 1, :]
                )

            accl = lax.fori_loop(
                0, _F // 2, lin_body, jnp.zeros((_D,), jnp.float32)
            )

            s = jnp.sum(acc) + jnp.sum(accl) * (1.0 / 16.0)
            e = bb * _G + g
            lane = lax.iota(jnp.int32, 16)
            plsc.store_scatter(
                res_v,
                [jnp.full((16,), e, dtype=jnp.int32)],
                jnp.full((16,), s, dtype=jnp.float32),
                mask=lane == 0,
            )

    build(0, 0)
    fire(0)

    def group_body(t, carry):
        b0 = t * 2
        build(b0 + 1, 1)
        fire(1)
        drain(0)
        compute(b0, 0)

        @pl.when(b0 + 2 < _NBLK)
        def _():
            build(b0 + 2, 0)
            fire(0)

        drain(1)
        compute(b0 + 1, 1)
        return carry

    lax.fori_loop(0, _NBLK // 2, group_body, 0)

    pltpu.sync_copy(res_v, out_hbm.at[pl.ds(base, _EPW)])


def kernel(x, E, w, b):
    xpad = jnp.pad(x.astype(jnp.int32), ((0, 0), (0, _S - _F))).reshape(-1)
    w16 = jnp.broadcast_to(w.reshape(_TOTAL, 1), (_TOTAL, _D))
    out = _ffm_sc(
        xpad, E, w16,
        jnp.asarray(_OFFT), jnp.asarray(_AROW), jnp.asarray(_BROW),
    )
    return out + b[0]


# speedup vs baseline: 2.0494x; 2.0494x over previous
import functools

import jax
import jax.numpy as jnp
import numpy as np
from jax import lax
from jax.experimental import pallas as pl
from jax.experimental.pallas import tpu as pltpu
from jax.experimental.pallas import tpu_sc as plsc

_F = 26
_FIELD = 1000
_TOTAL = _F * _FIELD
_D = 16
_B = 4096
_P = (_F * (_F - 1)) // 2
_PP = 336
_NIDX = 2 * _PP
_NCHUNK = _NIDX // 112
_NW = 32
_EPW = _B // _NW
_TAB = _NIDX + 32

_mesh = plsc.VectorSubcoreMesh(
    core_axis_name="c", subcore_axis_name="s", num_cores=2, num_subcores=16
)

_RC = 1000
_NCH = _TOTAL // _RC
_NU = _F * _NCH
_UPW = -(-_NU // _NW)


@functools.partial(
    pl.kernel,
    out_type=jax.ShapeDtypeStruct((_F * _TOTAL, _D), jnp.float32),
    mesh=_mesh,
    scratch_types=[
        pltpu.VMEM((2, _D, _RC), jnp.float32),
        pltpu.VMEM((2, _RC, _D), jnp.float32),
        pltpu.SemaphoreType.DMA,
        pltpu.SemaphoreType.DMA,
        pltpu.SemaphoreType.DMA,
        pltpu.SemaphoreType.DMA,
    ],
    compiler_params=pltpu.CompilerParams(
        needs_layout_passes=False, use_tc_tiling_on_sc=False
    ),
)
def _tr_sc(et_hbm, er_hbm, st_v, ot_v, si0, si1, so0, so1):
    wid = lax.axis_index("s") * 2 + lax.axis_index("c")
    sis = (si0, si1)
    sos = (so0, so1)
    rlane = lax.iota(jnp.int32, 16)

    def unit(t, buf):
        return wid + (t * 2 + buf) * _NW

    def stage(u, buf):
        j = u // _NCH
        r0 = (u % _NCH) * _RC
        pltpu.async_copy(
            et_hbm.at[j, :, pl.ds(r0, _RC)], st_v.at[buf], sis[buf]
        )

    def stage_wait(u, buf):
        j = u // _NCH
        r0 = (u % _NCH) * _RC
        pltpu.make_async_copy(
            et_hbm.at[j, :, pl.ds(r0, _RC)], st_v.at[buf], sis[buf]
        ).wait()

    def transpose(buf):
        def grp(rg, _):
            base = rg * _D
            ridx = rlane + base
            for d in range(_D):
                vals = st_v[buf, d, pl.ds(base, _D)]
                plsc.store_scatter(
                    ot_v.at[buf],
                    [ridx, jnp.full((16,), d, dtype=jnp.int32)],
                    vals,
                )
            return _

        lax.fori_loop(0, _RC // _D, grp, 0)
        base = _RC - _D
        ridx = rlane + base
        for d in range(_D):
            vals = st_v[buf, d, pl.ds(base, _D)]
            plsc.store_scatter(
                ot_v.at[buf],
                [ridx, jnp.full((16,), d, dtype=jnp.int32)],
                vals,
            )

    def flush(u, buf):
        j = u // _NCH
        r0 = (u % _NCH) * _RC
        pltpu.async_copy(
            ot_v.at[buf], er_hbm.at[pl.ds(j * _TOTAL + r0, _RC), :], sos[buf]
        ).wait()

    @pl.when(unit(0, 0) < _NU)
    def _():
        stage(unit(0, 0), 0)

    def step(t, carry):
        u0 = unit(t, 0)
        u1 = unit(t, 1)

        @pl.when(u1 < _NU)
        def _():
            stage(u1, 1)

        @pl.when(u0 < _NU)
        def _():
            stage_wait(u0, 0)
            transpose(0)
            flush(u0, 0)

        @pl.when(unit(t + 1, 0) < _NU)
        def _():
            stage(unit(t + 1, 0), 0)

        @pl.when(u1 < _NU)
        def _():
            stage_wait(u1, 1)
            transpose(1)
            flush(u1, 1)

        return carry

    lax.fori_loop(0, _UPW // 2, step, 0)


def _build_tables():
    fidx = np.zeros(_TAB, np.int32)
    cadd = np.zeros(_TAB, np.int32)
    p = 0
    for i in range(_F - 1):
        for j in range(i + 1, _F):
            fidx[p] = i
            cadd[p] = j * _TOTAL + i * _FIELD
            fidx[_PP + p] = j
            cadd[_PP + p] = i * _TOTAL + j * _FIELD
            p += 1
    for q in range(_F):
        fidx[_NIDX + q] = q
        cadd[_NIDX + q] = q * _FIELD
    mask = np.zeros(_D, np.float32)
    mask[: _F - 16] = 1.0
    return fidx, cadd, mask


_TF, _TC, _LMASK = _build_tables()


@functools.partial(
    pl.kernel,
    out_type=jax.ShapeDtypeStruct((_B,), jnp.float32),
    mesh=_mesh,
    scratch_types=[
        pltpu.VMEM((_EPW * _F,), jnp.int32),
        pltpu.VMEM((_TOTAL,), jnp.float32),
        pltpu.VMEM((_TAB,), jnp.int32),
        pltpu.VMEM((_TAB,), jnp.int32),
        pltpu.VMEM((_D,), jnp.float32),
        pltpu.VMEM((_NIDX,), jnp.int32),
        pltpu.VMEM((_NIDX,), jnp.int32),
        pltpu.VMEM((2, _NIDX, _D), jnp.float32),
        pltpu.VMEM((_EPW,), jnp.float32),
        pltpu.SemaphoreType.DMA,
        pltpu.SemaphoreType.DMA,
    ],
    compiler_params=pltpu.CompilerParams(
        needs_layout_passes=False, use_tc_tiling_on_sc=False
    ),
)
def _ffm_sc(x_hbm, e_hbm, w_hbm, tf_hbm, tc_hbm, lm_hbm, out_hbm,
            x_v, w_v, tf_v, tc_v, lm_v, idx0_v, idx1_v, rows_v, res_v,
            sem0, sem1):
    wid = lax.axis_index("s") * 2 + lax.axis_index("c")
    base = wid * _EPW

    pltpu.sync_copy(x_hbm.at[pl.ds(base * _F, _EPW * _F)], x_v)
    pltpu.sync_copy(w_hbm, w_v)
    pltpu.sync_copy(tf_hbm, tf_v)
    pltpu.sync_copy(tc_hbm, tc_v)
    pltpu.sync_copy(lm_hbm, lm_v)

    sems = (sem0, sem1)
    idxs = (idx0_v, idx1_v)

    def build_indices(e, buf):
        xbase = e * _F
        ib = idxs[buf]
        for k in range(_NIDX // _D):
            fv = tf_v[pl.ds(k * _D, _D)]
            cv = tc_v[pl.ds(k * _D, _D)]
            xi = plsc.load_gather(x_v, [fv + xbase])
            ib[pl.ds(k * _D, _D)] = xi + cv

    def fire(buf):
        sem = sems[buf]
        ib = idxs[buf]
        for c in range(_NCHUNK):
            pltpu.async_copy(
                e_hbm.at[ib.at[pl.ds(c * 112, 112)]],
                rows_v.at[buf, pl.ds(c * 112, 112)],
                sem,
            )

    def drain(buf):
        sem = sems[buf]
        ib = idxs[buf]
        for c in range(_NCHUNK):
            pltpu.make_async_copy(
                e_hbm.at[ib.at[pl.ds(c * 112, 112)]],
                rows_v.at[buf, pl.ds(c * 112, 112)],
                sem,
            ).wait()

    def compute(e, buf):
        def pair_body(p, acc):
            q = p * 5
            for u in range(5):
                acc = acc + rows_v[buf, q + u, :] * rows_v[buf, _PP + q + u, :]
            return acc
        acc = lax.fori_loop(0, _P // 5, pair_body, jnp.zeros((_D,), jnp.float32))

        xbase = e * _F
        fv0 = tf_v[pl.ds(_NIDX, _D)]
        cv0 = tc_v[pl.ds(_NIDX, _D)]
        xi0 = plsc.load_gather(x_v, [fv0 + xbase])
        l0 = plsc.load_gather(w_v, [xi0 + cv0])
        fv1 = tf_v[pl.ds(_NIDX + _D, _D)]
        cv1 = tc_v[pl.ds(_NIDX + _D, _D)]
        xi1 = plsc.load_gather(x_v, [fv1 + xbase])
        l1 = plsc.load_gather(w_v, [xi1 + cv1]) * lm_v[...]

        s = jnp.sum(acc + l0 + l1)
        lane = lax.iota(jnp.int32, 16)
        ev = jnp.full((16,), e, dtype=jnp.int32)
        sv = jnp.full((16,), s, dtype=jnp.float32)
        plsc.store_scatter(res_v, [ev], sv, mask=lane == 0)

    build_indices(0, 0)
    fire(0)

    def group_body(t, carry):
        e0 = t * 2
        build_indices(e0 + 1, 1)
        fire(1)
        drain(0)
        compute(e0, 0)

        @pl.when(e0 + 2 < _EPW)
        def _():
            build_indices(e0 + 2, 0)
            fire(0)

        drain(1)
        compute(e0 + 1, 1)
        return carry

    lax.fori_loop(0, _EPW // 2, group_body, 0)

    pltpu.sync_copy(res_v, out_hbm.at[pl.ds(base, _EPW)])


def kernel(x, E, w, b):
    xf = x.reshape(-1).astype(jnp.int32)
    wf = w.reshape(-1)
    et = jnp.transpose(E, (0, 2, 1))
    er = _tr_sc(et)
    out = _ffm_sc(xf, er, wf, jnp.asarray(_TF), jnp.asarray(_TC),
                  jnp.asarray(_LMASK))
    return out + b[0]
